# Initial kernel scaffold; baseline (speedup 1.0000x reference)
import jax
import jax.numpy as jnp
from jax.experimental import pallas as pl


def kernel(x, FL_weight, FL_bias, FE_weight, offsets):
    raise NotImplementedError("write your pallas kernel here")


# speedup vs baseline: 28.2688x; 28.2688x over previous
import functools

import jax
import jax.numpy as jnp
from jax import lax
from jax.experimental import pallas as pl
from jax.experimental.pallas import tpu as pltpu
from jax.experimental.pallas import tpu_sc as plsc

B = 16384
F = 26
D = 128
V = 26000
NC = 2
NS = 16
NW = NC * NS
RPW = B // NW
CR = 8
NCHUNK = RPW // CR
IPC = CR * F
HALF = IPC // 2
LANES = 16
DCH = D // LANES


def _fm_body(xi_hbm, fl_hbm, bias_hbm, fe_hbm, out_hbm,
             xi_v, fl_v, bias_v, ebuf0, ebuf1, zbuf, ybuf, obuf,
             sem0, sem1):
    wid = lax.axis_index("s") * NC + lax.axis_index("c")
    base = wid * RPW
    ibase = base * F

    pltpu.sync_copy(xi_hbm.at[pl.ds(ibase, RPW * F)], xi_v)
    pltpu.sync_copy(fl_hbm, fl_v)
    pltpu.sync_copy(bias_hbm, bias_v)

    def fire(chunk, ebuf, sem):
        for h in range(2):
            idx = xi_v.at[pl.ds(chunk * IPC + h * HALF, HALF)]
            pltpu.async_copy(fe_hbm.at[idx], ebuf.at[pl.ds(h * HALF, HALF)],
                             sem)

    def drain(ebuf, sem):
        for h in range(2):
            idx = xi_v.at[pl.ds(h * HALF, HALF)]
            pltpu.make_async_copy(fe_hbm.at[idx],
                                  ebuf.at[pl.ds(h * HALF, HALF)], sem).wait()

    fire(0, ebuf0, sem0)
    fire(1, ebuf1, sem1)

    iota = lax.iota(jnp.int32, LANES)
    stride = iota * F

    def fl_group(g, _):
        def fl_field(f, acc):
            addr = stride + (g * (LANES * F) + f)
            xiv = plsc.load_gather(xi_v, [addr])
            return acc + plsc.load_gather(fl_v, [xiv])

        flacc = lax.fori_loop(0, F, fl_field,
                              jnp.zeros((LANES,), jnp.float32))
        zbuf[pl.ds(g * LANES, LANES)] = flacc
        return 0

    lax.fori_loop(0, RPW // LANES, fl_group, 0)

    def row_body(r, carry, chunk, ebuf):
        del carry
        def field_pair(fi, carry):
            s = list(carry[:DCH])
            q = carry[DCH]
            for u in range(2):
                row = r * F + 2 * fi + u
                for c in range(DCH):
                    v = ebuf[row, pl.ds(c * LANES, LANES)]
                    s[c] = s[c] + v
                    q = q + v * v
            return (*s, q)

        init = tuple(jnp.zeros((LANES,), jnp.float32)
                     for _ in range(DCH + 1))
        acc = lax.fori_loop(0, F // 2, field_pair, init)
        s2 = acc[0] * acc[0]
        for c in range(1, DCH):
            s2 = s2 + acc[c] * acc[c]
        rowv = s2 - acc[DCH]
        rowtot = plsc.cumsum(0.5 * rowv)
        idxv = jnp.full((LANES,), chunk * CR + r, jnp.int32)
        plsc.store_scatter(ybuf, [idxv], rowtot, mask=iota == (LANES - 1))
        return 0

    def pair_body(p, _):
        for bsel in range(2):
            chunk = 2 * p + bsel
            ebuf = ebuf0 if bsel == 0 else ebuf1
            sem = sem0 if bsel == 0 else sem1
            drain(ebuf, sem)
            lax.fori_loop(0, CR, functools.partial(row_body, chunk=chunk,
                                                   ebuf=ebuf), 0)

            @pl.when(p < NCHUNK // 2 - 1)
            def _():
                fire(chunk + 2, ebuf, sem)
        return 0

    lax.fori_loop(0, NCHUNK // 2, pair_body, 0)

    biasv = bias_v[...]

    def out_group(g, _):
        zv = zbuf[pl.ds(g * LANES, LANES)] + ybuf[pl.ds(g * LANES, LANES)]
        zv = zv + biasv
        obuf[pl.ds(g * LANES, LANES)] = 1.0 / (1.0 + jnp.exp(-zv))
        return 0

    lax.fori_loop(0, RPW // LANES, out_group, 0)
    pltpu.sync_copy(obuf, out_hbm.at[pl.ds(base, RPW)])


@jax.jit
def _fm_sc(xi_flat, fl, bias16, fe):
    mesh = plsc.VectorSubcoreMesh(core_axis_name="c", subcore_axis_name="s")
    run = pl.kernel(
        _fm_body,
        mesh=mesh,
        compiler_params=pltpu.CompilerParams(needs_layout_passes=False),
        out_type=jax.ShapeDtypeStruct((B,), jnp.float32),
        scratch_types=[
            pltpu.VMEM((RPW * F,), jnp.int32),
            pltpu.VMEM((V,), jnp.float32),
            pltpu.VMEM((LANES,), jnp.float32),
            pltpu.VMEM((IPC, D), jnp.float32),
            pltpu.VMEM((IPC, D), jnp.float32),
            pltpu.VMEM((RPW,), jnp.float32),
            pltpu.VMEM((RPW,), jnp.float32),
            pltpu.VMEM((RPW,), jnp.float32),
            pltpu.SemaphoreType.DMA,
            pltpu.SemaphoreType.DMA,
        ],
    )
    return run(xi_flat, fl, bias16, fe)


def kernel(x, FL_weight, FL_bias, FE_weight, offsets):
    xi_flat = (x + offsets[None, :]).reshape(-1).astype(jnp.int32)
    fl = FL_weight[:, 0]
    bias16 = jnp.broadcast_to(FL_bias.astype(jnp.float32), (LANES,))
    return _fm_sc(xi_flat, fl, bias16, FE_weight)
